# Initial kernel scaffold; baseline (speedup 1.0000x reference)
import jax
import jax.numpy as jnp
from jax.experimental import pallas as pl


def kernel(x, edge_index, batch, W0, b0, W1, b1, W2, b2, Wm1, bm1, Wm2, bm2):
    raise NotImplementedError("write your pallas kernel here")


# speedup vs baseline: 6.2978x; 6.2978x over previous
import functools

import jax
import jax.numpy as jnp
from jax import lax
from jax.experimental import pallas as pl
from jax.experimental.pallas import tpu as pltpu
from jax.experimental.pallas import tpu_sc as plsc

N = 10000
NUM_GRAPHS = 64
E = 320000
E_TOT = E + N
CHUNK = 128
NCH = 162
E_PAD = 16 * NCH * CHUNK
DUMMY = 10100
NPAD = 10240
ROWS_PER_TILE = NPAD // 16
BN = 400
NBLK = N // BN
FW = 64

_mesh = plsc.VectorSubcoreMesh(core_axis_name="c", subcore_axis_name="s",
                               num_cores=2, num_subcores=16)


@functools.partial(
    pl.kernel,
    out_type=jax.ShapeDtypeStruct((2, NPAD, 16), jnp.float32),
    mesh=_mesh,
    scratch_types=[
        pltpu.VMEM((NCH, CHUNK), jnp.int32),
        pltpu.VMEM((CHUNK, 16), jnp.float32),
        pltpu.VMEM((ROWS_PER_TILE, 16), jnp.float32),
        pltpu.VMEM_SHARED((NPAD, 16), jnp.float32),
    ],
    compiler_params=pltpu.CompilerParams(use_tc_tiling_on_sc=False),
)
def _sc_hist(idx_hbm, out_hbm, idx_v, ones_v, buf_v, acc_sh):
    c = lax.axis_index("c")
    s = lax.axis_index("s")
    one = jnp.full((16,), 1.0, jnp.float32)
    zero = jnp.zeros((16,), jnp.float32)

    def fill_ones(i, _):
        ones_v[i] = one
        return 0

    lax.fori_loop(0, CHUNK, fill_ones, 0)

    def fill_zero(i, _):
        buf_v[i] = zero
        return 0

    lax.fori_loop(0, ROWS_PER_TILE, fill_zero, 0)
    pltpu.sync_copy(buf_v, acc_sh.at[pl.ds(s * ROWS_PER_TILE, ROWS_PER_TILE)])
    plsc.subcore_barrier()

    pltpu.sync_copy(idx_hbm.at[c].at[s], idx_v)

    def body(j, _):
        pltpu.sync_copy(ones_v, acc_sh.at[idx_v.at[j]], add=True)
        return 0

    lax.fori_loop(0, NCH, body, 0)
    plsc.subcore_barrier()

    pltpu.sync_copy(acc_sh.at[pl.ds(s * ROWS_PER_TILE, ROWS_PER_TILE)], buf_v)
    pltpu.sync_copy(buf_v, out_hbm.at[c].at[pl.ds(s * ROWS_PER_TILE, ROWS_PER_TILE)])


@functools.partial(
    pl.kernel,
    out_type=jax.ShapeDtypeStruct((4, NPAD, FW), jnp.float32),
    mesh=_mesh,
    scratch_types=[
        pltpu.VMEM((NCH, CHUNK), jnp.int32),
        pltpu.VMEM((NCH, CHUNK), jnp.int32),
        pltpu.VMEM((CHUNK, FW), jnp.float32),
        pltpu.VMEM_SHARED((NPAD, FW), jnp.float32),
        pltpu.SemaphoreType.DMA,
    ],
    compiler_params=pltpu.CompilerParams(use_tc_tiling_on_sc=False),
)
def _sc_agg(g_hbm, rowidx_hbm, colidx_hbm, zeros_hbm, out_hbm,
            ridx_v, cidx_v, rows_v, acc_sh, sem):
    c = lax.axis_index("c")
    s = lax.axis_index("s")
    base = s * ROWS_PER_TILE

    pltpu.sync_copy(rowidx_hbm.at[s], ridx_v)
    pltpu.sync_copy(colidx_hbm.at[s], cidx_v)

    for p in range(2):
        q = c * 2 + p
        pltpu.sync_copy(zeros_hbm, rows_v)
        for k in range(ROWS_PER_TILE // CHUNK):
            pltpu.sync_copy(rows_v, acc_sh.at[pl.ds(base + k * CHUNK, CHUNK)])
        plsc.subcore_barrier()

        tbl = g_hbm.at[q]

        def body(j, _):
            pltpu.async_copy(tbl.at[ridx_v.at[j]], rows_v, sem).wait()
            pltpu.sync_copy(rows_v, acc_sh.at[cidx_v.at[j]], add=True)
            return 0

        lax.fori_loop(0, NCH, body, 0)
        plsc.subcore_barrier()

        for k in range(ROWS_PER_TILE // CHUNK):
            pltpu.sync_copy(acc_sh.at[pl.ds(base + k * CHUNK, CHUNK)], rows_v)
            pltpu.sync_copy(rows_v, out_hbm.at[q].at[pl.ds(base + k * CHUNK, CHUNK)])
        plsc.subcore_barrier()


def _tc_in_body(x_ref, w_ref, b_ref, hist_ref, g_ref, dis_ref, sdst_ref):
    deg = hist_ref[0, :, 0:1]
    cnt = hist_ref[1, :, 0:1]
    dis = lax.rsqrt(jnp.maximum(deg, 1.0))
    lin = jnp.dot(x_ref[...], w_ref[0], preferred_element_type=jnp.float32)
    g_ref[0] = dis * (lin + b_ref[0])
    dis_ref[...] = dis
    sdst_ref[...] = dis / jnp.maximum(cnt, 1.0)


def _tc_in(x, w, b, hist):
    return pl.pallas_call(
        _tc_in_body,
        grid=(NBLK, 4),
        in_specs=[
            pl.BlockSpec((BN, 128), lambda i, q: (i, 0)),
            pl.BlockSpec((1, 128, FW), lambda i, q: (q, 0, 0)),
            pl.BlockSpec((1, 1, FW), lambda i, q: (q, 0, 0)),
            pl.BlockSpec((2, BN, 16), lambda i, q: (0, i, 0)),
        ],
        out_specs=[
            pl.BlockSpec((1, BN, FW), lambda i, q: (q, i, 0)),
            pl.BlockSpec((BN, 1), lambda i, q: (i, 0)),
            pl.BlockSpec((BN, 1), lambda i, q: (i, 0)),
        ],
        out_shape=[
            jax.ShapeDtypeStruct((4, N, FW), jnp.float32),
            jax.ShapeDtypeStruct((N, 1), jnp.float32),
            jax.ShapeDtypeStruct((N, 1), jnp.float32),
        ],
    )(x, w, b, hist)


def _tc_mid_body(agg_ref, sdst_ref, dis_ref, w_ref, b_ref, g_ref):
    ci = pl.program_id(2)
    a = jnp.maximum(sdst_ref[...] * agg_ref[0], 0.0)
    part = jnp.dot(a, w_ref[0, 0], preferred_element_type=jnp.float32)

    @pl.when(ci == 0)
    def _():
        g_ref[0] = part + b_ref[0]

    @pl.when(jnp.logical_and(ci > 0, ci < 3))
    def _():
        g_ref[0] = g_ref[0] + part

    @pl.when(ci == 3)
    def _():
        g_ref[0] = dis_ref[...] * (g_ref[0] + part)


def _tc_mid(agg, sdst, dis, w, b):
    return pl.pallas_call(
        _tc_mid_body,
        grid=(NBLK, 4, 4),
        in_specs=[
            pl.BlockSpec((1, BN, FW), lambda i, qo, ci: (ci, i, 0)),
            pl.BlockSpec((BN, 1), lambda i, qo, ci: (i, 0)),
            pl.BlockSpec((BN, 1), lambda i, qo, ci: (i, 0)),
            pl.BlockSpec((1, 1, FW, FW), lambda i, qo, ci: (ci, qo, 0, 0)),
            pl.BlockSpec((1, 1, FW), lambda i, qo, ci: (qo, 0, 0)),
        ],
        out_specs=pl.BlockSpec((1, BN, FW), lambda i, qo, ci: (qo, i, 0)),
        out_shape=jax.ShapeDtypeStruct((4, N, FW), jnp.float32),
    )(agg, sdst, dis, w, b)


def _tc_fin_body(agg_ref, sdst_ref, batch_ref, wm1_ref, bm1_ref,
                 wm2_ref, bm2_ref, out_ref, pool_acc, cnt_acc):
    i = pl.program_id(0)
    hs = [jnp.maximum(sdst_ref[...] * agg_ref[q], 0.0) for q in range(4)]
    bt = batch_ref[0]
    oh = (bt == lax.broadcasted_iota(jnp.int32, (NUM_GRAPHS, BN), 0))
    oh = oh.astype(jnp.float32)

    @pl.when(i == 0)
    def _():
        pool_acc[...] = jnp.zeros_like(pool_acc)
        cnt_acc[...] = jnp.zeros_like(cnt_acc)

    for q in range(4):
        pool_acc[:, q * FW:(q + 1) * FW] += jnp.dot(
            oh, hs[q], preferred_element_type=jnp.float32)
    cnt_acc[...] += jnp.sum(oh, axis=1, keepdims=True)

    @pl.when(i == NBLK - 1)
    def _():
        pooled = pool_acc[...] / jnp.maximum(cnt_acc[...], 1.0)
        z = jnp.dot(pooled, wm1_ref[...], preferred_element_type=jnp.float32)
        z = jnp.maximum(z + bm1_ref[...], 0.0)
        out_ref[...] = (
            jnp.dot(z, wm2_ref[...], preferred_element_type=jnp.float32)
            + bm2_ref[...]
        )


def _tc_fin(agg, sdst, batch3, wm1, bm1, wm2, bm2):
    return pl.pallas_call(
        _tc_fin_body,
        grid=(NBLK,),
        in_specs=[
            pl.BlockSpec((4, BN, FW), lambda i: (0, i, 0)),
            pl.BlockSpec((BN, 1), lambda i: (i, 0)),
            pl.BlockSpec((1, 1, BN), lambda i: (i, 0, 0)),
            pl.BlockSpec((256, 256), lambda i: (0, 0)),
            pl.BlockSpec((1, 256), lambda i: (0, 0)),
            pl.BlockSpec((256, 64), lambda i: (0, 0)),
            pl.BlockSpec((1, 64), lambda i: (0, 0)),
        ],
        out_specs=pl.BlockSpec((NUM_GRAPHS, 64), lambda i: (0, 0)),
        out_shape=jax.ShapeDtypeStruct((NUM_GRAPHS, 64), jnp.float32),
        scratch_shapes=[
            pltpu.VMEM((NUM_GRAPHS, 256), jnp.float32),
            pltpu.VMEM((NUM_GRAPHS, 1), jnp.float32),
        ],
    )(agg, sdst, batch3, wm1, bm1, wm2, bm2)


def kernel(x, edge_index, batch, W0, b0, W1, b1, W2, b2, Wm1, bm1, Wm2, bm2):
    sl = jnp.arange(N, dtype=jnp.int32)
    row = jnp.concatenate([edge_index[0], sl])
    col = jnp.concatenate([edge_index[1], sl])
    npad = E_PAD - E_TOT
    pad_dummy = jnp.full((npad,), DUMMY, jnp.int32)
    row_g = jnp.concatenate([row, jnp.zeros((npad,), jnp.int32)])
    row_h = jnp.concatenate([row, pad_dummy])
    col_p = jnp.concatenate([col, pad_dummy])

    hist_idx = jnp.stack([row_h.reshape(16, NCH, CHUNK),
                          col_p.reshape(16, NCH, CHUNK)])
    hist = _sc_hist(hist_idx)

    rowidx = row_g.reshape(16, NCH, CHUNK)
    colidx = col_p.reshape(16, NCH, CHUNK)
    zeros128 = jnp.zeros((CHUNK, FW), jnp.float32)

    w0r = W0.reshape(128, 4, FW).transpose(1, 0, 2)
    g1, dis, sdst = _tc_in(x, w0r, b0.reshape(4, 1, FW), hist[:, :N, :])

    agg1 = _sc_agg(g1, rowidx, colidx, zeros128)[:, :N, :]
    w1r = W1.reshape(4, FW, 4, FW).transpose(0, 2, 1, 3)
    g2 = _tc_mid(agg1, sdst, dis, w1r, b1.reshape(4, 1, FW))

    agg2 = _sc_agg(g2, rowidx, colidx, zeros128)[:, :N, :]
    w2r = W2.reshape(4, FW, 4, FW).transpose(0, 2, 1, 3)
    g3 = _tc_mid(agg2, sdst, dis, w2r, b2.reshape(4, 1, FW))

    agg3 = _sc_agg(g3, rowidx, colidx, zeros128)[:, :N, :]
    batch3 = batch.reshape(NBLK, 1, BN)
    return _tc_fin(agg3, sdst, batch3, Wm1, bm1.reshape(1, 256),
                   Wm2, bm2.reshape(1, 64))


# speedup vs baseline: 8.4254x; 1.3378x over previous
import functools

import jax
import jax.numpy as jnp
from jax import lax
from jax.experimental import pallas as pl
from jax.experimental.pallas import tpu as pltpu
from jax.experimental.pallas import tpu_sc as plsc

N = 10000
NUM_GRAPHS = 64
E = 320000
E_TOT = E + N
CHUNK = 128
NCH = 162
E_PAD = 16 * NCH * CHUNK
DUMMY = 10100
NPAD = 10240
ROWS_PER_TILE = NPAD // 16
BN = 400
NBLK = N // BN
FW = 64

_mesh = plsc.VectorSubcoreMesh(core_axis_name="c", subcore_axis_name="s",
                               num_cores=2, num_subcores=16)


@functools.partial(
    pl.kernel,
    out_type=jax.ShapeDtypeStruct((2, NPAD, 16), jnp.float32),
    mesh=_mesh,
    scratch_types=[
        pltpu.VMEM((NCH, CHUNK), jnp.int32),
        pltpu.VMEM((CHUNK, 16), jnp.float32),
        pltpu.VMEM((ROWS_PER_TILE, 16), jnp.float32),
        pltpu.VMEM_SHARED((NPAD, 16), jnp.float32),
    ],
    compiler_params=pltpu.CompilerParams(use_tc_tiling_on_sc=False),
)
def _sc_hist(idx_hbm, out_hbm, idx_v, ones_v, buf_v, acc_sh):
    c = lax.axis_index("c")
    s = lax.axis_index("s")
    one = jnp.full((16,), 1.0, jnp.float32)
    zero = jnp.zeros((16,), jnp.float32)

    def fill_ones(i, _):
        ones_v[i] = one
        return 0

    lax.fori_loop(0, CHUNK, fill_ones, 0)

    def fill_zero(i, _):
        buf_v[i] = zero
        return 0

    lax.fori_loop(0, ROWS_PER_TILE, fill_zero, 0)
    pltpu.sync_copy(buf_v, acc_sh.at[pl.ds(s * ROWS_PER_TILE, ROWS_PER_TILE)])
    plsc.subcore_barrier()

    pltpu.sync_copy(idx_hbm.at[c].at[s], idx_v)

    def body(j, _):
        pltpu.sync_copy(ones_v, acc_sh.at[idx_v.at[j]], add=True)
        return 0

    lax.fori_loop(0, NCH, body, 0)
    plsc.subcore_barrier()

    pltpu.sync_copy(acc_sh.at[pl.ds(s * ROWS_PER_TILE, ROWS_PER_TILE)], buf_v)
    pltpu.sync_copy(buf_v, out_hbm.at[c].at[pl.ds(s * ROWS_PER_TILE, ROWS_PER_TILE)])


@functools.partial(
    pl.kernel,
    out_type=jax.ShapeDtypeStruct((4, NPAD, FW), jnp.float32),
    mesh=_mesh,
    scratch_types=[
        pltpu.VMEM((NCH, CHUNK), jnp.int32),
        pltpu.VMEM((NCH, CHUNK), jnp.int32),
        pltpu.VMEM((CHUNK, FW), jnp.float32),
        pltpu.VMEM((CHUNK, FW), jnp.float32),
        pltpu.VMEM_SHARED((NPAD, FW), jnp.float32),
        pltpu.SemaphoreType.DMA,
        pltpu.SemaphoreType.DMA,
    ],
    compiler_params=pltpu.CompilerParams(use_tc_tiling_on_sc=False),
)
def _sc_agg(g_hbm, rowidx_hbm, colidx_hbm, zeros_hbm, out_hbm,
            ridx_v, cidx_v, rows0_v, rows1_v, acc_sh, sem0, sem1):
    c = lax.axis_index("c")
    s = lax.axis_index("s")
    base = s * ROWS_PER_TILE

    pltpu.sync_copy(rowidx_hbm.at[s], ridx_v)
    pltpu.sync_copy(colidx_hbm.at[s], cidx_v)

    for p in range(2):
        q = c * 2 + p
        pltpu.sync_copy(zeros_hbm, rows0_v)
        for k in range(ROWS_PER_TILE // CHUNK):
            pltpu.sync_copy(rows0_v, acc_sh.at[pl.ds(base + k * CHUNK, CHUNK)])
        plsc.subcore_barrier()

        tbl = g_hbm.at[q]

        pltpu.async_copy(tbl.at[ridx_v.at[0]], rows0_v, sem0)

        def body(jj, _):
            j = jj * 2
            pltpu.async_copy(tbl.at[ridx_v.at[j + 1]], rows1_v, sem1)
            pltpu.make_async_copy(tbl.at[ridx_v.at[j]], rows0_v, sem0).wait()
            pltpu.sync_copy(rows0_v, acc_sh.at[cidx_v.at[j]], add=True)
            nxt = jnp.minimum(j + 2, NCH - 1)
            pltpu.async_copy(tbl.at[ridx_v.at[nxt]], rows0_v, sem0)
            pltpu.make_async_copy(tbl.at[ridx_v.at[j + 1]], rows1_v, sem1).wait()
            pltpu.sync_copy(rows1_v, acc_sh.at[cidx_v.at[j + 1]], add=True)
            return 0

        lax.fori_loop(0, NCH // 2, body, 0)
        pltpu.make_async_copy(tbl.at[ridx_v.at[NCH - 1]], rows0_v, sem0).wait()
        plsc.subcore_barrier()

        for k in range(ROWS_PER_TILE // CHUNK):
            pltpu.sync_copy(acc_sh.at[pl.ds(base + k * CHUNK, CHUNK)], rows0_v)
            pltpu.sync_copy(rows0_v, out_hbm.at[q].at[pl.ds(base + k * CHUNK, CHUNK)])
        plsc.subcore_barrier()


def _tc_in_body(x_ref, w_ref, b_ref, hist_ref, g_ref, dis_ref, sdst_ref):
    deg = hist_ref[0, :, 0:1]
    cnt = hist_ref[1, :, 0:1]
    dis = lax.rsqrt(jnp.maximum(deg, 1.0))
    lin = jnp.dot(x_ref[...], w_ref[0], preferred_element_type=jnp.float32)
    g_ref[0] = dis * (lin + b_ref[0])
    dis_ref[...] = dis
    sdst_ref[...] = dis / jnp.maximum(cnt, 1.0)


def _tc_in(x, w, b, hist):
    return pl.pallas_call(
        _tc_in_body,
        grid=(NBLK, 4),
        in_specs=[
            pl.BlockSpec((BN, 128), lambda i, q: (i, 0)),
            pl.BlockSpec((1, 128, FW), lambda i, q: (q, 0, 0)),
            pl.BlockSpec((1, 1, FW), lambda i, q: (q, 0, 0)),
            pl.BlockSpec((2, BN, 16), lambda i, q: (0, i, 0)),
        ],
        out_specs=[
            pl.BlockSpec((1, BN, FW), lambda i, q: (q, i, 0)),
            pl.BlockSpec((BN, 1), lambda i, q: (i, 0)),
            pl.BlockSpec((BN, 1), lambda i, q: (i, 0)),
        ],
        out_shape=[
            jax.ShapeDtypeStruct((4, N, FW), jnp.float32),
            jax.ShapeDtypeStruct((N, 1), jnp.float32),
            jax.ShapeDtypeStruct((N, 1), jnp.float32),
        ],
    )(x, w, b, hist)


def _tc_mid_body(agg_ref, sdst_ref, dis_ref, w_ref, b_ref, g_ref):
    ci = pl.program_id(2)
    a = jnp.maximum(sdst_ref[...] * agg_ref[0], 0.0)
    part = jnp.dot(a, w_ref[0, 0], preferred_element_type=jnp.float32)

    @pl.when(ci == 0)
    def _():
        g_ref[0] = part + b_ref[0]

    @pl.when(jnp.logical_and(ci > 0, ci < 3))
    def _():
        g_ref[0] = g_ref[0] + part

    @pl.when(ci == 3)
    def _():
        g_ref[0] = dis_ref[...] * (g_ref[0] + part)


def _tc_mid(agg, sdst, dis, w, b):
    return pl.pallas_call(
        _tc_mid_body,
        grid=(NBLK, 4, 4),
        in_specs=[
            pl.BlockSpec((1, BN, FW), lambda i, qo, ci: (ci, i, 0)),
            pl.BlockSpec((BN, 1), lambda i, qo, ci: (i, 0)),
            pl.BlockSpec((BN, 1), lambda i, qo, ci: (i, 0)),
            pl.BlockSpec((1, 1, FW, FW), lambda i, qo, ci: (ci, qo, 0, 0)),
            pl.BlockSpec((1, 1, FW), lambda i, qo, ci: (qo, 0, 0)),
        ],
        out_specs=pl.BlockSpec((1, BN, FW), lambda i, qo, ci: (qo, i, 0)),
        out_shape=jax.ShapeDtypeStruct((4, N, FW), jnp.float32),
    )(agg, sdst, dis, w, b)


def _tc_fin_body(agg_ref, sdst_ref, batch_ref, wm1_ref, bm1_ref,
                 wm2_ref, bm2_ref, out_ref, pool_acc, cnt_acc):
    i = pl.program_id(0)
    hs = [jnp.maximum(sdst_ref[...] * agg_ref[q], 0.0) for q in range(4)]
    bt = batch_ref[0]
    oh = (bt == lax.broadcasted_iota(jnp.int32, (NUM_GRAPHS, BN), 0))
    oh = oh.astype(jnp.float32)

    @pl.when(i == 0)
    def _():
        pool_acc[...] = jnp.zeros_like(pool_acc)
        cnt_acc[...] = jnp.zeros_like(cnt_acc)

    for q in range(4):
        pool_acc[:, q * FW:(q + 1) * FW] += jnp.dot(
            oh, hs[q], preferred_element_type=jnp.float32)
    cnt_acc[...] += jnp.sum(oh, axis=1, keepdims=True)

    @pl.when(i == NBLK - 1)
    def _():
        pooled = pool_acc[...] / jnp.maximum(cnt_acc[...], 1.0)
        z = jnp.dot(pooled, wm1_ref[...], preferred_element_type=jnp.float32)
        z = jnp.maximum(z + bm1_ref[...], 0.0)
        out_ref[...] = (
            jnp.dot(z, wm2_ref[...], preferred_element_type=jnp.float32)
            + bm2_ref[...]
        )


def _tc_fin(agg, sdst, batch3, wm1, bm1, wm2, bm2):
    return pl.pallas_call(
        _tc_fin_body,
        grid=(NBLK,),
        in_specs=[
            pl.BlockSpec((4, BN, FW), lambda i: (0, i, 0)),
            pl.BlockSpec((BN, 1), lambda i: (i, 0)),
            pl.BlockSpec((1, 1, BN), lambda i: (i, 0, 0)),
            pl.BlockSpec((256, 256), lambda i: (0, 0)),
            pl.BlockSpec((1, 256), lambda i: (0, 0)),
            pl.BlockSpec((256, 64), lambda i: (0, 0)),
            pl.BlockSpec((1, 64), lambda i: (0, 0)),
        ],
        out_specs=pl.BlockSpec((NUM_GRAPHS, 64), lambda i: (0, 0)),
        out_shape=jax.ShapeDtypeStruct((NUM_GRAPHS, 64), jnp.float32),
        scratch_shapes=[
            pltpu.VMEM((NUM_GRAPHS, 256), jnp.float32),
            pltpu.VMEM((NUM_GRAPHS, 1), jnp.float32),
        ],
    )(agg, sdst, batch3, wm1, bm1, wm2, bm2)


def kernel(x, edge_index, batch, W0, b0, W1, b1, W2, b2, Wm1, bm1, Wm2, bm2):
    sl = jnp.arange(N, dtype=jnp.int32)
    row = jnp.concatenate([edge_index[0], sl])
    col = jnp.concatenate([edge_index[1], sl])
    npad = E_PAD - E_TOT
    pad_dummy = jnp.full((npad,), DUMMY, jnp.int32)
    row_g = jnp.concatenate([row, jnp.zeros((npad,), jnp.int32)])
    row_h = jnp.concatenate([row, pad_dummy])
    col_p = jnp.concatenate([col, pad_dummy])

    hist_idx = jnp.stack([row_h.reshape(16, NCH, CHUNK),
                          col_p.reshape(16, NCH, CHUNK)])
    hist = _sc_hist(hist_idx)

    rowidx = row_g.reshape(16, NCH, CHUNK)
    colidx = col_p.reshape(16, NCH, CHUNK)
    zeros128 = jnp.zeros((CHUNK, FW), jnp.float32)

    w0r = W0.reshape(128, 4, FW).transpose(1, 0, 2)
    g1, dis, sdst = _tc_in(x, w0r, b0.reshape(4, 1, FW), hist)

    agg1 = _sc_agg(g1, rowidx, colidx, zeros128)
    w1r = W1.reshape(4, FW, 4, FW).transpose(0, 2, 1, 3)
    g2 = _tc_mid(agg1, sdst, dis, w1r, b1.reshape(4, 1, FW))

    agg2 = _sc_agg(g2, rowidx, colidx, zeros128)
    w2r = W2.reshape(4, FW, 4, FW).transpose(0, 2, 1, 3)
    g3 = _tc_mid(agg2, sdst, dis, w2r, b2.reshape(4, 1, FW))

    agg3 = _sc_agg(g3, rowidx, colidx, zeros128)
    batch3 = batch.reshape(NBLK, 1, BN)
    return _tc_fin(agg3, sdst, batch3, Wm1, bm1.reshape(1, 256),
                   Wm2, bm2.reshape(1, 64))


# speedup vs baseline: 11.2514x; 1.3354x over previous
import functools

import jax
import jax.numpy as jnp
from jax import lax
from jax.experimental import pallas as pl
from jax.experimental.pallas import tpu as pltpu
from jax.experimental.pallas import tpu_sc as plsc

N = 10000
NUM_GRAPHS = 64
E = 320000
E_TOT = E + N
CHUNK = 128
NCH = 162
E_PAD = 16 * NCH * CHUNK
DUMMY = 10100
NPAD = 10240
ROWS_PER_TILE = NPAD // 16
BN = 400
NBLK = N // BN
FW = 64
D_HID = 256

_mesh = plsc.VectorSubcoreMesh(core_axis_name="c", subcore_axis_name="s",
                               num_cores=2, num_subcores=16)


@functools.partial(
    pl.kernel,
    out_type=jax.ShapeDtypeStruct((2, NPAD, 16), jnp.float32),
    mesh=_mesh,
    scratch_types=[
        pltpu.VMEM((NCH, CHUNK), jnp.int32),
        pltpu.VMEM((CHUNK, 16), jnp.float32),
        pltpu.VMEM((ROWS_PER_TILE, 16), jnp.float32),
        pltpu.VMEM_SHARED((NPAD, 16), jnp.float32),
    ],
    compiler_params=pltpu.CompilerParams(use_tc_tiling_on_sc=False),
)
def _sc_hist(idx_hbm, out_hbm, idx_v, ones_v, buf_v, acc_sh):
    c = lax.axis_index("c")
    s = lax.axis_index("s")
    one = jnp.full((16,), 1.0, jnp.float32)
    zero = jnp.zeros((16,), jnp.float32)

    def fill_ones(i, _):
        ones_v[i] = one
        return 0

    lax.fori_loop(0, CHUNK, fill_ones, 0)

    def fill_zero(i, _):
        buf_v[i] = zero
        return 0

    lax.fori_loop(0, ROWS_PER_TILE, fill_zero, 0)
    pltpu.sync_copy(buf_v, acc_sh.at[pl.ds(s * ROWS_PER_TILE, ROWS_PER_TILE)])
    plsc.subcore_barrier()

    pltpu.sync_copy(idx_hbm.at[c].at[s], idx_v)

    def body(j, _):
        pltpu.sync_copy(ones_v, acc_sh.at[idx_v.at[j]], add=True)
        return 0

    lax.fori_loop(0, NCH, body, 0)
    plsc.subcore_barrier()

    pltpu.sync_copy(acc_sh.at[pl.ds(s * ROWS_PER_TILE, ROWS_PER_TILE)], buf_v)
    pltpu.sync_copy(buf_v, out_hbm.at[c].at[pl.ds(s * ROWS_PER_TILE, ROWS_PER_TILE)])


@functools.partial(
    pl.kernel,
    out_type=jax.ShapeDtypeStruct((NPAD, 4, FW), jnp.float32),
    mesh=_mesh,
    scratch_types=[
        pltpu.VMEM((2, NCH, CHUNK), jnp.int32),
        pltpu.VMEM((NCH, CHUNK), jnp.int32),
        pltpu.VMEM((CHUNK, FW), jnp.float32),
        pltpu.VMEM((CHUNK, FW), jnp.float32),
        pltpu.VMEM_SHARED((NPAD, FW), jnp.float32),
        pltpu.SemaphoreType.DMA,
        pltpu.SemaphoreType.DMA,
    ],
    compiler_params=pltpu.CompilerParams(use_tc_tiling_on_sc=False),
)
def _sc_agg(g4_hbm, rowidx4_hbm, colidx_hbm, zeros_hbm, out_hbm,
            ridx_v, cidx_v, rows0_v, rows1_v, acc_sh, sem0, sem1):
    c = lax.axis_index("c")
    s = lax.axis_index("s")
    base = s * ROWS_PER_TILE

    pltpu.sync_copy(colidx_hbm.at[s], cidx_v)

    for p in range(2):
        q = c * 2 + p
        pltpu.sync_copy(rowidx4_hbm.at[q].at[s], ridx_v.at[p])

        pltpu.sync_copy(zeros_hbm, rows0_v)
        for k in range(ROWS_PER_TILE // CHUNK):
            pltpu.sync_copy(rows0_v, acc_sh.at[pl.ds(base + k * CHUNK, CHUNK)])
        plsc.subcore_barrier()

        ridx_p = ridx_v.at[p]

        pltpu.async_copy(g4_hbm.at[ridx_p.at[0]], rows0_v, sem0)

        def body(jj, _):
            j = jj * 2
            pltpu.async_copy(g4_hbm.at[ridx_p.at[j + 1]], rows1_v, sem1)
            pltpu.make_async_copy(g4_hbm.at[ridx_p.at[j]], rows0_v, sem0).wait()
            pltpu.sync_copy(rows0_v, acc_sh.at[cidx_v.at[j]], add=True)
            nxt = jnp.minimum(j + 2, NCH - 1)
            pltpu.async_copy(g4_hbm.at[ridx_p.at[nxt]], rows0_v, sem0)
            pltpu.make_async_copy(g4_hbm.at[ridx_p.at[j + 1]], rows1_v, sem1).wait()
            pltpu.sync_copy(rows1_v, acc_sh.at[cidx_v.at[j + 1]], add=True)
            return 0

        lax.fori_loop(0, NCH // 2, body, 0)
        pltpu.make_async_copy(g4_hbm.at[ridx_p.at[NCH - 1]], rows0_v, sem0).wait()
        plsc.subcore_barrier()

        for k in range(ROWS_PER_TILE // CHUNK):
            pltpu.sync_copy(acc_sh.at[pl.ds(base + k * CHUNK, CHUNK)], rows0_v)
            pltpu.sync_copy(rows0_v, out_hbm.at[pl.ds(base + k * CHUNK, CHUNK), q])
        plsc.subcore_barrier()


def _tc_in_body(x_ref, w_ref, b_ref, hist_ref, g_ref, dis_ref, sdst_ref):
    deg = hist_ref[0, :, 0:1]
    cnt = hist_ref[1, :, 0:1]
    dis = lax.rsqrt(jnp.maximum(deg, 1.0))
    lin = jnp.dot(x_ref[...], w_ref[...], preferred_element_type=jnp.float32)
    g_ref[...] = dis * (lin + b_ref[...])
    dis_ref[...] = dis
    sdst_ref[...] = dis / jnp.maximum(cnt, 1.0)


def _tc_in(x, w, b, hist):
    return pl.pallas_call(
        _tc_in_body,
        grid=(NBLK,),
        in_specs=[
            pl.BlockSpec((BN, 128), lambda i: (i, 0)),
            pl.BlockSpec((128, D_HID), lambda i: (0, 0)),
            pl.BlockSpec((1, D_HID), lambda i: (0, 0)),
            pl.BlockSpec((2, BN, 16), lambda i: (0, i, 0)),
        ],
        out_specs=[
            pl.BlockSpec((BN, D_HID), lambda i: (i, 0)),
            pl.BlockSpec((BN, 1), lambda i: (i, 0)),
            pl.BlockSpec((BN, 1), lambda i: (i, 0)),
        ],
        out_shape=[
            jax.ShapeDtypeStruct((N, D_HID), jnp.float32),
            jax.ShapeDtypeStruct((N, 1), jnp.float32),
            jax.ShapeDtypeStruct((N, 1), jnp.float32),
        ],
    )(x, w, b, hist)


def _tc_mid_body(agg_ref, sdst_ref, dis_ref, w_ref, b_ref, g_ref):
    h = jnp.maximum(sdst_ref[...] * agg_ref[...], 0.0)
    lin = jnp.dot(h, w_ref[...], preferred_element_type=jnp.float32)
    g_ref[...] = dis_ref[...] * (lin + b_ref[...])


def _tc_mid(agg2d, sdst, dis, w, b):
    return pl.pallas_call(
        _tc_mid_body,
        grid=(NBLK,),
        in_specs=[
            pl.BlockSpec((BN, D_HID), lambda i: (i, 0)),
            pl.BlockSpec((BN, 1), lambda i: (i, 0)),
            pl.BlockSpec((BN, 1), lambda i: (i, 0)),
            pl.BlockSpec((D_HID, D_HID), lambda i: (0, 0)),
            pl.BlockSpec((1, D_HID), lambda i: (0, 0)),
        ],
        out_specs=pl.BlockSpec((BN, D_HID), lambda i: (i, 0)),
        out_shape=jax.ShapeDtypeStruct((N, D_HID), jnp.float32),
    )(agg2d, sdst, dis, w, b)


def _tc_fin_body(agg_ref, sdst_ref, batch_ref, wm1_ref, bm1_ref,
                 wm2_ref, bm2_ref, out_ref, pool_acc, cnt_acc):
    i = pl.program_id(0)
    h = jnp.maximum(sdst_ref[...] * agg_ref[...], 0.0)
    bt = batch_ref[0]
    oh = (bt == lax.broadcasted_iota(jnp.int32, (NUM_GRAPHS, BN), 0))
    oh = oh.astype(jnp.float32)

    @pl.when(i == 0)
    def _():
        pool_acc[...] = jnp.zeros_like(pool_acc)
        cnt_acc[...] = jnp.zeros_like(cnt_acc)

    pool_acc[...] += jnp.dot(oh, h, preferred_element_type=jnp.float32)
    cnt_acc[...] += jnp.sum(oh, axis=1, keepdims=True)

    @pl.when(i == NBLK - 1)
    def _():
        pooled = pool_acc[...] / jnp.maximum(cnt_acc[...], 1.0)
        z = jnp.dot(pooled, wm1_ref[...], preferred_element_type=jnp.float32)
        z = jnp.maximum(z + bm1_ref[...], 0.0)
        out_ref[...] = (
            jnp.dot(z, wm2_ref[...], preferred_element_type=jnp.float32)
            + bm2_ref[...]
        )


def _tc_fin(agg2d, sdst, batch3, wm1, bm1, wm2, bm2):
    return pl.pallas_call(
        _tc_fin_body,
        grid=(NBLK,),
        in_specs=[
            pl.BlockSpec((BN, D_HID), lambda i: (i, 0)),
            pl.BlockSpec((BN, 1), lambda i: (i, 0)),
            pl.BlockSpec((1, 1, BN), lambda i: (i, 0, 0)),
            pl.BlockSpec((D_HID, D_HID), lambda i: (0, 0)),
            pl.BlockSpec((1, D_HID), lambda i: (0, 0)),
            pl.BlockSpec((D_HID, 64), lambda i: (0, 0)),
            pl.BlockSpec((1, 64), lambda i: (0, 0)),
        ],
        out_specs=pl.BlockSpec((NUM_GRAPHS, 64), lambda i: (0, 0)),
        out_shape=jax.ShapeDtypeStruct((NUM_GRAPHS, 64), jnp.float32),
        scratch_shapes=[
            pltpu.VMEM((NUM_GRAPHS, D_HID), jnp.float32),
            pltpu.VMEM((NUM_GRAPHS, 1), jnp.float32),
        ],
    )(agg2d, sdst, batch3, wm1, bm1, wm2, bm2)


def kernel(x, edge_index, batch, W0, b0, W1, b1, W2, b2, Wm1, bm1, Wm2, bm2):
    sl = jnp.arange(N, dtype=jnp.int32)
    row = jnp.concatenate([edge_index[0], sl])
    col = jnp.concatenate([edge_index[1], sl])
    npad = E_PAD - E_TOT
    pad_dummy = jnp.full((npad,), DUMMY, jnp.int32)
    row_g = jnp.concatenate([row, jnp.zeros((npad,), jnp.int32)])
    row_h = jnp.concatenate([row, pad_dummy])
    col_p = jnp.concatenate([col, pad_dummy])

    hist_idx = jnp.stack([row_h.reshape(16, NCH, CHUNK),
                          col_p.reshape(16, NCH, CHUNK)])
    hist = _sc_hist(hist_idx)

    row4 = row_g.reshape(1, 16, NCH, CHUNK) * 4
    rowidx4 = row4 + jnp.arange(4, dtype=jnp.int32).reshape(4, 1, 1, 1)
    colidx = col_p.reshape(16, NCH, CHUNK)
    zeros_blk = jnp.zeros((CHUNK, FW), jnp.float32)

    def agg(g2d):
        out = _sc_agg(g2d.reshape(4 * N, FW), rowidx4, colidx, zeros_blk)
        return out.reshape(NPAD, 4 * FW)

    g1, dis, sdst = _tc_in(x, W0, b0.reshape(1, D_HID), hist)
    g2 = _tc_mid(agg(g1), sdst, dis, W1, b1.reshape(1, D_HID))
    g3 = _tc_mid(agg(g2), sdst, dis, W2, b2.reshape(1, D_HID))
    agg3 = agg(g3)
    batch3 = batch.reshape(NBLK, 1, BN)
    return _tc_fin(agg3, sdst, batch3, Wm1, bm1.reshape(1, D_HID),
                   Wm2, bm2.reshape(1, 64))


# speedup vs baseline: 11.3830x; 1.0117x over previous
import functools

import jax
import jax.numpy as jnp
from jax import lax
from jax.experimental import pallas as pl
from jax.experimental.pallas import tpu as pltpu
from jax.experimental.pallas import tpu_sc as plsc

N = 10000
NUM_GRAPHS = 64
E = 320000
E_TOT = E + N
CHUNK = 128
NCH = 162
E_PAD = 16 * NCH * CHUNK
DUMMY = 10100
NPAD = 10240
ROWS_PER_TILE = NPAD // 16
BN = 400
NBLK = N // BN
FW = 64
D_HID = 256

_mesh = plsc.VectorSubcoreMesh(core_axis_name="c", subcore_axis_name="s",
                               num_cores=2, num_subcores=16)


@functools.partial(
    pl.kernel,
    out_type=jax.ShapeDtypeStruct((2, NPAD, 16), jnp.float32),
    mesh=_mesh,
    scratch_types=[
        pltpu.VMEM((NCH, CHUNK), jnp.int32),
        pltpu.VMEM((CHUNK, 16), jnp.float32),
        pltpu.VMEM((ROWS_PER_TILE, 16), jnp.float32),
        pltpu.VMEM_SHARED((NPAD, 16), jnp.float32),
    ],
    compiler_params=pltpu.CompilerParams(use_tc_tiling_on_sc=False),
)
def _sc_hist(idx_hbm, out_hbm, idx_v, ones_v, buf_v, acc_sh):
    c = lax.axis_index("c")
    s = lax.axis_index("s")
    one = jnp.full((16,), 1.0, jnp.float32)
    zero = jnp.zeros((16,), jnp.float32)

    def fill_ones(i, _):
        ones_v[i] = one
        return 0

    lax.fori_loop(0, CHUNK, fill_ones, 0)

    def fill_zero(i, _):
        buf_v[i] = zero
        return 0

    lax.fori_loop(0, ROWS_PER_TILE, fill_zero, 0)
    pltpu.sync_copy(buf_v, acc_sh.at[pl.ds(s * ROWS_PER_TILE, ROWS_PER_TILE)])
    plsc.subcore_barrier()

    pltpu.sync_copy(idx_hbm.at[c].at[s], idx_v)

    def body(j, _):
        pltpu.sync_copy(ones_v, acc_sh.at[idx_v.at[j]], add=True)
        return 0

    lax.fori_loop(0, NCH, body, 0)
    plsc.subcore_barrier()

    pltpu.sync_copy(acc_sh.at[pl.ds(s * ROWS_PER_TILE, ROWS_PER_TILE)], buf_v)
    pltpu.sync_copy(buf_v, out_hbm.at[c].at[pl.ds(s * ROWS_PER_TILE, ROWS_PER_TILE)])


@functools.partial(
    pl.kernel,
    out_type=jax.ShapeDtypeStruct((NPAD, 4, FW), jnp.float32),
    mesh=_mesh,
    scratch_types=[
        pltpu.VMEM((2, NCH, CHUNK), jnp.int32),
        pltpu.VMEM((NCH, CHUNK), jnp.int32),
        pltpu.VMEM((3, CHUNK, FW), jnp.float32),
        pltpu.VMEM_SHARED((NPAD, FW), jnp.float32),
        pltpu.SemaphoreType.DMA,
        pltpu.SemaphoreType.DMA,
    ],
    compiler_params=pltpu.CompilerParams(use_tc_tiling_on_sc=False),
)
def _sc_agg(g4_hbm, rowidx4_hbm, colidx_hbm, zeros_hbm, out_hbm,
            ridx_v, cidx_v, bufs_v, acc_sh, semg1, sems1):
    semg = [semg1] * 3
    sems = [sems1] * 3
    c = lax.axis_index("c")
    s = lax.axis_index("s")
    base = s * ROWS_PER_TILE
    NB = 3
    NROUND = NCH // NB
    REM = NCH - NROUND * NB

    pltpu.sync_copy(colidx_hbm.at[s], cidx_v)

    for p in range(2):
        q = c * 2 + p
        pltpu.sync_copy(rowidx4_hbm.at[q].at[s], ridx_v.at[p])

        pltpu.sync_copy(zeros_hbm, bufs_v.at[0])
        for k in range(ROWS_PER_TILE // CHUNK):
            pltpu.async_copy(bufs_v.at[0],
                             acc_sh.at[pl.ds(base + k * CHUNK, CHUNK)], sems[0])
        for k in range(ROWS_PER_TILE // CHUNK):
            pltpu.make_async_copy(
                bufs_v.at[0],
                acc_sh.at[pl.ds(base + k * CHUNK, CHUNK)], sems[0]).wait()
        plsc.subcore_barrier()

        ridx_p = ridx_v.at[p]

        def gather(j, b):
            pltpu.async_copy(g4_hbm.at[ridx_p.at[j]], bufs_v.at[b], semg[b])

        def gather_wait(j, b):
            pltpu.make_async_copy(g4_hbm.at[ridx_p.at[j]], bufs_v.at[b],
                                  semg[b]).wait()

        def scatter(j, b):
            pltpu.async_copy(bufs_v.at[b], acc_sh.at[cidx_v.at[j]], sems[b],
                             add=True)

        def scatter_wait(j, b):
            pltpu.make_async_copy(bufs_v.at[b], acc_sh.at[cidx_v.at[j]],
                                  sems[b]).wait()

        for b in range(NB):
            gather(b, b)

        def body(jj, _):
            j = jj * NB
            for b in range(NB):
                gather_wait(j + b, b)
                scatter(j + b, b)
            for b in range(NB):
                scatter_wait(j + b, b)
                nxt = jnp.minimum(j + NB + b, NCH - 1)
                gather(nxt, b)
            return 0

        lax.fori_loop(0, NROUND, body, 0)
        for b in range(NB):
            j = NROUND * NB + b
            if b < REM:
                gather_wait(j, b)
                scatter(j, b)
                scatter_wait(j, b)
            else:
                gather_wait(NCH - 1, b)
        plsc.subcore_barrier()

        NK = ROWS_PER_TILE // CHUNK
        for k in range(NK):
            b = k % NB
            if k >= NB:
                pltpu.make_async_copy(
                    bufs_v.at[b],
                    out_hbm.at[pl.ds(base + (k - NB) * CHUNK, CHUNK), q],
                    sems[b]).wait()
            pltpu.async_copy(acc_sh.at[pl.ds(base + k * CHUNK, CHUNK)],
                             bufs_v.at[b], semg[b])
            pltpu.make_async_copy(acc_sh.at[pl.ds(base + k * CHUNK, CHUNK)],
                                  bufs_v.at[b], semg[b]).wait()
            pltpu.async_copy(bufs_v.at[b],
                             out_hbm.at[pl.ds(base + k * CHUNK, CHUNK), q],
                             sems[b])
        for k in range(max(NK - NB, 0), NK):
            b = k % NB
            pltpu.make_async_copy(
                bufs_v.at[b],
                out_hbm.at[pl.ds(base + k * CHUNK, CHUNK), q], sems[b]).wait()
        plsc.subcore_barrier()


def _tc_in_body(x_ref, w_ref, b_ref, hist_ref, g_ref, dis_ref, sdst_ref):
    deg = hist_ref[0, :, 0:1]
    cnt = hist_ref[1, :, 0:1]
    dis = lax.rsqrt(jnp.maximum(deg, 1.0))
    lin = jnp.dot(x_ref[...], w_ref[...], preferred_element_type=jnp.float32)
    g_ref[...] = dis * (lin + b_ref[...])
    dis_ref[...] = dis
    sdst_ref[...] = dis / jnp.maximum(cnt, 1.0)


def _tc_in(x, w, b, hist):
    return pl.pallas_call(
        _tc_in_body,
        grid=(NBLK,),
        in_specs=[
            pl.BlockSpec((BN, 128), lambda i: (i, 0)),
            pl.BlockSpec((128, D_HID), lambda i: (0, 0)),
            pl.BlockSpec((1, D_HID), lambda i: (0, 0)),
            pl.BlockSpec((2, BN, 16), lambda i: (0, i, 0)),
        ],
        out_specs=[
            pl.BlockSpec((BN, D_HID), lambda i: (i, 0)),
            pl.BlockSpec((BN, 1), lambda i: (i, 0)),
            pl.BlockSpec((BN, 1), lambda i: (i, 0)),
        ],
        out_shape=[
            jax.ShapeDtypeStruct((N, D_HID), jnp.float32),
            jax.ShapeDtypeStruct((N, 1), jnp.float32),
            jax.ShapeDtypeStruct((N, 1), jnp.float32),
        ],
    )(x, w, b, hist)


def _tc_mid_body(agg_ref, sdst_ref, dis_ref, w_ref, b_ref, g_ref):
    h = jnp.maximum(sdst_ref[...] * agg_ref[...], 0.0)
    lin = jnp.dot(h, w_ref[...], preferred_element_type=jnp.float32)
    g_ref[...] = dis_ref[...] * (lin + b_ref[...])


def _tc_mid(agg2d, sdst, dis, w, b):
    return pl.pallas_call(
        _tc_mid_body,
        grid=(NBLK,),
        in_specs=[
            pl.BlockSpec((BN, D_HID), lambda i: (i, 0)),
            pl.BlockSpec((BN, 1), lambda i: (i, 0)),
            pl.BlockSpec((BN, 1), lambda i: (i, 0)),
            pl.BlockSpec((D_HID, D_HID), lambda i: (0, 0)),
            pl.BlockSpec((1, D_HID), lambda i: (0, 0)),
        ],
        out_specs=pl.BlockSpec((BN, D_HID), lambda i: (i, 0)),
        out_shape=jax.ShapeDtypeStruct((N, D_HID), jnp.float32),
    )(agg2d, sdst, dis, w, b)


def _tc_fin_body(agg_ref, sdst_ref, batch_ref, wm1_ref, bm1_ref,
                 wm2_ref, bm2_ref, out_ref, pool_acc, cnt_acc):
    i = pl.program_id(0)
    h = jnp.maximum(sdst_ref[...] * agg_ref[...], 0.0)
    bt = batch_ref[0]
    oh = (bt == lax.broadcasted_iota(jnp.int32, (NUM_GRAPHS, BN), 0))
    oh = oh.astype(jnp.float32)

    @pl.when(i == 0)
    def _():
        pool_acc[...] = jnp.zeros_like(pool_acc)
        cnt_acc[...] = jnp.zeros_like(cnt_acc)

    pool_acc[...] += jnp.dot(oh, h, preferred_element_type=jnp.float32)
    cnt_acc[...] += jnp.sum(oh, axis=1, keepdims=True)

    @pl.when(i == NBLK - 1)
    def _():
        pooled = pool_acc[...] / jnp.maximum(cnt_acc[...], 1.0)
        z = jnp.dot(pooled, wm1_ref[...], preferred_element_type=jnp.float32)
        z = jnp.maximum(z + bm1_ref[...], 0.0)
        out_ref[...] = (
            jnp.dot(z, wm2_ref[...], preferred_element_type=jnp.float32)
            + bm2_ref[...]
        )


def _tc_fin(agg2d, sdst, batch3, wm1, bm1, wm2, bm2):
    return pl.pallas_call(
        _tc_fin_body,
        grid=(NBLK,),
        in_specs=[
            pl.BlockSpec((BN, D_HID), lambda i: (i, 0)),
            pl.BlockSpec((BN, 1), lambda i: (i, 0)),
            pl.BlockSpec((1, 1, BN), lambda i: (i, 0, 0)),
            pl.BlockSpec((D_HID, D_HID), lambda i: (0, 0)),
            pl.BlockSpec((1, D_HID), lambda i: (0, 0)),
            pl.BlockSpec((D_HID, 64), lambda i: (0, 0)),
            pl.BlockSpec((1, 64), lambda i: (0, 0)),
        ],
        out_specs=pl.BlockSpec((NUM_GRAPHS, 64), lambda i: (0, 0)),
        out_shape=jax.ShapeDtypeStruct((NUM_GRAPHS, 64), jnp.float32),
        scratch_shapes=[
            pltpu.VMEM((NUM_GRAPHS, D_HID), jnp.float32),
            pltpu.VMEM((NUM_GRAPHS, 1), jnp.float32),
        ],
    )(agg2d, sdst, batch3, wm1, bm1, wm2, bm2)


def kernel(x, edge_index, batch, W0, b0, W1, b1, W2, b2, Wm1, bm1, Wm2, bm2):
    sl = jnp.arange(N, dtype=jnp.int32)
    row = jnp.concatenate([edge_index[0], sl])
    col = jnp.concatenate([edge_index[1], sl])
    npad = E_PAD - E_TOT
    pad_dummy = jnp.full((npad,), DUMMY, jnp.int32)
    row_g = jnp.concatenate([row, jnp.zeros((npad,), jnp.int32)])
    row_h = jnp.concatenate([row, pad_dummy])
    col_p = jnp.concatenate([col, pad_dummy])

    hist_idx = jnp.stack([row_h.reshape(16, NCH, CHUNK),
                          col_p.reshape(16, NCH, CHUNK)])
    hist = _sc_hist(hist_idx)

    row4 = row_g.reshape(1, 16, NCH, CHUNK) * 4
    rowidx4 = row4 + jnp.arange(4, dtype=jnp.int32).reshape(4, 1, 1, 1)
    colidx = col_p.reshape(16, NCH, CHUNK)
    zeros_blk = jnp.zeros((CHUNK, FW), jnp.float32)

    def agg(g2d):
        out = _sc_agg(g2d.reshape(4 * N, FW), rowidx4, colidx, zeros_blk)
        return out.reshape(NPAD, 4 * FW)

    g1, dis, sdst = _tc_in(x, W0, b0.reshape(1, D_HID), hist)
    g2 = _tc_mid(agg(g1), sdst, dis, W1, b1.reshape(1, D_HID))
    g3 = _tc_mid(agg(g2), sdst, dis, W2, b2.reshape(1, D_HID))
    agg3 = agg(g3)
    batch3 = batch.reshape(NBLK, 1, BN)
    return _tc_fin(agg3, sdst, batch3, Wm1, bm1.reshape(1, D_HID),
                   Wm2, bm2.reshape(1, 64))


# speedup vs baseline: 16.8875x; 1.4836x over previous
import functools

import jax
import jax.numpy as jnp
from jax import lax
from jax.experimental import pallas as pl
from jax.experimental.pallas import tpu as pltpu
from jax.experimental.pallas import tpu_sc as plsc

N = 10000
NUM_GRAPHS = 64
E = 320000
E_TOT = E + N
CHUNK = 128
NCH = 162
E_PAD = 16 * NCH * CHUNK
DUMMY = 10100
NPAD = 10240
ROWS_PER_TILE = NPAD // 16
BN = 400
NBLK = N // BN
FW = 64
D_HID = 256

_mesh = plsc.VectorSubcoreMesh(core_axis_name="c", subcore_axis_name="s",
                               num_cores=2, num_subcores=16)


@functools.partial(
    pl.kernel,
    out_type=jax.ShapeDtypeStruct((2, NPAD, 16), jnp.float32),
    mesh=_mesh,
    scratch_types=[
        pltpu.VMEM((NCH, CHUNK), jnp.int32),
        pltpu.VMEM((CHUNK, 16), jnp.float32),
        pltpu.VMEM((ROWS_PER_TILE, 16), jnp.float32),
        pltpu.VMEM_SHARED((NPAD, 16), jnp.float32),
    ],
    compiler_params=pltpu.CompilerParams(use_tc_tiling_on_sc=False),
)
def _sc_hist(idx_hbm, out_hbm, idx_v, ones_v, buf_v, acc_sh):
    c = lax.axis_index("c")
    s = lax.axis_index("s")
    one = jnp.full((16,), 1.0, jnp.float32)
    zero = jnp.zeros((16,), jnp.float32)

    def fill_ones(i, _):
        ones_v[i] = one
        return 0

    lax.fori_loop(0, CHUNK, fill_ones, 0)

    def fill_zero(i, _):
        buf_v[i] = zero
        return 0

    lax.fori_loop(0, ROWS_PER_TILE, fill_zero, 0)
    pltpu.sync_copy(buf_v, acc_sh.at[pl.ds(s * ROWS_PER_TILE, ROWS_PER_TILE)])
    plsc.subcore_barrier()

    pltpu.sync_copy(idx_hbm.at[c].at[s], idx_v)

    def body(j, _):
        pltpu.sync_copy(ones_v, acc_sh.at[idx_v.at[j]], add=True)
        return 0

    lax.fori_loop(0, NCH, body, 0)
    plsc.subcore_barrier()

    pltpu.sync_copy(acc_sh.at[pl.ds(s * ROWS_PER_TILE, ROWS_PER_TILE)], buf_v)
    pltpu.sync_copy(buf_v, out_hbm.at[c].at[pl.ds(s * ROWS_PER_TILE, ROWS_PER_TILE)])


@functools.partial(
    pl.kernel,
    out_type=jax.ShapeDtypeStruct((NPAD, 4, FW), jnp.bfloat16),
    mesh=_mesh,
    scratch_types=[
        pltpu.VMEM((2, NCH, CHUNK), jnp.int32),
        pltpu.VMEM((NCH, CHUNK), jnp.int32),
        pltpu.VMEM((3, CHUNK, FW), jnp.bfloat16),
        pltpu.VMEM_SHARED((NPAD, FW), jnp.bfloat16),
        pltpu.SemaphoreType.DMA,
        pltpu.SemaphoreType.DMA,
    ],
    compiler_params=pltpu.CompilerParams(use_tc_tiling_on_sc=False),
)
def _sc_agg(g4_hbm, rowidx4_hbm, colidx_hbm, zeros_hbm, out_hbm,
            ridx_v, cidx_v, bufs_v, acc_sh, semg1, sems1):
    semg = [semg1] * 3
    sems = [sems1] * 3
    c = lax.axis_index("c")
    s = lax.axis_index("s")
    base = s * ROWS_PER_TILE
    NB = 3
    NROUND = NCH // NB
    REM = NCH - NROUND * NB

    pltpu.sync_copy(colidx_hbm.at[s], cidx_v)

    for p in range(2):
        q = c * 2 + p
        pltpu.sync_copy(rowidx4_hbm.at[q].at[s], ridx_v.at[p])

        pltpu.sync_copy(zeros_hbm, bufs_v.at[0])
        for k in range(ROWS_PER_TILE // CHUNK):
            pltpu.async_copy(bufs_v.at[0],
                             acc_sh.at[pl.ds(base + k * CHUNK, CHUNK)], sems[0])
        for k in range(ROWS_PER_TILE // CHUNK):
            pltpu.make_async_copy(
                bufs_v.at[0],
                acc_sh.at[pl.ds(base + k * CHUNK, CHUNK)], sems[0]).wait()
        plsc.subcore_barrier()

        ridx_p = ridx_v.at[p]

        def gather(j, b):
            pltpu.async_copy(g4_hbm.at[ridx_p.at[j]], bufs_v.at[b], semg[b])

        def gather_wait(j, b):
            pltpu.make_async_copy(g4_hbm.at[ridx_p.at[j]], bufs_v.at[b],
                                  semg[b]).wait()

        def scatter(j, b):
            pltpu.async_copy(bufs_v.at[b], acc_sh.at[cidx_v.at[j]], sems[b],
                             add=True)

        def scatter_wait(j, b):
            pltpu.make_async_copy(bufs_v.at[b], acc_sh.at[cidx_v.at[j]],
                                  sems[b]).wait()

        for b in range(NB):
            gather(b, b)

        def body(jj, _):
            j = jj * NB
            for b in range(NB):
                gather_wait(j + b, b)
                scatter(j + b, b)
            for b in range(NB):
                scatter_wait(j + b, b)
                nxt = jnp.minimum(j + NB + b, NCH - 1)
                gather(nxt, b)
            return 0

        lax.fori_loop(0, NROUND, body, 0)
        for b in range(NB):
            j = NROUND * NB + b
            if b < REM:
                gather_wait(j, b)
                scatter(j, b)
                scatter_wait(j, b)
            else:
                gather_wait(NCH - 1, b)
        plsc.subcore_barrier()

        NK = ROWS_PER_TILE // CHUNK
        for k in range(NK):
            b = k % NB
            if k >= NB:
                pltpu.make_async_copy(
                    bufs_v.at[b],
                    out_hbm.at[pl.ds(base + (k - NB) * CHUNK, CHUNK), q],
                    sems[b]).wait()
            pltpu.async_copy(acc_sh.at[pl.ds(base + k * CHUNK, CHUNK)],
                             bufs_v.at[b], semg[b])
            pltpu.make_async_copy(acc_sh.at[pl.ds(base + k * CHUNK, CHUNK)],
                                  bufs_v.at[b], semg[b]).wait()
            pltpu.async_copy(bufs_v.at[b],
                             out_hbm.at[pl.ds(base + k * CHUNK, CHUNK), q],
                             sems[b])
        for k in range(max(NK - NB, 0), NK):
            b = k % NB
            pltpu.make_async_copy(
                bufs_v.at[b],
                out_hbm.at[pl.ds(base + k * CHUNK, CHUNK), q], sems[b]).wait()
        plsc.subcore_barrier()


def _tc_in_body(x_ref, w_ref, b_ref, hist_ref, g_ref, dis_ref, sdst_ref):
    deg = hist_ref[0, :, 0:1]
    cnt = hist_ref[1, :, 0:1]
    dis = lax.rsqrt(jnp.maximum(deg, 1.0))
    lin = jnp.dot(x_ref[...], w_ref[...], preferred_element_type=jnp.float32)
    g_ref[...] = (dis * (lin + b_ref[...])).astype(jnp.bfloat16)
    dis_ref[...] = dis
    sdst_ref[...] = dis / jnp.maximum(cnt, 1.0)


def _tc_in(x, w, b, hist):
    return pl.pallas_call(
        _tc_in_body,
        grid=(NBLK,),
        in_specs=[
            pl.BlockSpec((BN, 128), lambda i: (i, 0)),
            pl.BlockSpec((128, D_HID), lambda i: (0, 0)),
            pl.BlockSpec((1, D_HID), lambda i: (0, 0)),
            pl.BlockSpec((2, BN, 16), lambda i: (0, i, 0)),
        ],
        out_specs=[
            pl.BlockSpec((BN, D_HID), lambda i: (i, 0)),
            pl.BlockSpec((BN, 1), lambda i: (i, 0)),
            pl.BlockSpec((BN, 1), lambda i: (i, 0)),
        ],
        out_shape=[
            jax.ShapeDtypeStruct((N, D_HID), jnp.bfloat16),
            jax.ShapeDtypeStruct((N, 1), jnp.float32),
            jax.ShapeDtypeStruct((N, 1), jnp.float32),
        ],
    )(x, w, b, hist)


def _tc_mid_body(agg_ref, sdst_ref, dis_ref, w_ref, b_ref, g_ref):
    a = agg_ref[...].astype(jnp.float32)
    h = jnp.maximum(sdst_ref[...] * a, 0.0)
    lin = jnp.dot(h, w_ref[...], preferred_element_type=jnp.float32)
    g_ref[...] = (dis_ref[...] * (lin + b_ref[...])).astype(jnp.bfloat16)


def _tc_mid(agg2d, sdst, dis, w, b):
    return pl.pallas_call(
        _tc_mid_body,
        grid=(NBLK,),
        in_specs=[
            pl.BlockSpec((BN, D_HID), lambda i: (i, 0)),
            pl.BlockSpec((BN, 1), lambda i: (i, 0)),
            pl.BlockSpec((BN, 1), lambda i: (i, 0)),
            pl.BlockSpec((D_HID, D_HID), lambda i: (0, 0)),
            pl.BlockSpec((1, D_HID), lambda i: (0, 0)),
        ],
        out_specs=pl.BlockSpec((BN, D_HID), lambda i: (i, 0)),
        out_shape=jax.ShapeDtypeStruct((N, D_HID), jnp.bfloat16),
    )(agg2d, sdst, dis, w, b)


def _tc_fin_body(agg_ref, sdst_ref, batch_ref, wm1_ref, bm1_ref,
                 wm2_ref, bm2_ref, out_ref, pool_acc, cnt_acc):
    i = pl.program_id(0)
    h = jnp.maximum(sdst_ref[...] * agg_ref[...].astype(jnp.float32), 0.0)
    bt = batch_ref[0]
    oh = (bt == lax.broadcasted_iota(jnp.int32, (NUM_GRAPHS, BN), 0))
    oh = oh.astype(jnp.float32)

    @pl.when(i == 0)
    def _():
        pool_acc[...] = jnp.zeros_like(pool_acc)
        cnt_acc[...] = jnp.zeros_like(cnt_acc)

    pool_acc[...] += jnp.dot(oh, h, preferred_element_type=jnp.float32)
    cnt_acc[...] += jnp.sum(oh, axis=1, keepdims=True)

    @pl.when(i == NBLK - 1)
    def _():
        pooled = pool_acc[...] / jnp.maximum(cnt_acc[...], 1.0)
        z = jnp.dot(pooled, wm1_ref[...], preferred_element_type=jnp.float32)
        z = jnp.maximum(z + bm1_ref[...], 0.0)
        out_ref[...] = (
            jnp.dot(z, wm2_ref[...], preferred_element_type=jnp.float32)
            + bm2_ref[...]
        )


def _tc_fin(agg2d, sdst, batch3, wm1, bm1, wm2, bm2):
    return pl.pallas_call(
        _tc_fin_body,
        grid=(NBLK,),
        in_specs=[
            pl.BlockSpec((BN, D_HID), lambda i: (i, 0)),
            pl.BlockSpec((BN, 1), lambda i: (i, 0)),
            pl.BlockSpec((1, 1, BN), lambda i: (i, 0, 0)),
            pl.BlockSpec((D_HID, D_HID), lambda i: (0, 0)),
            pl.BlockSpec((1, D_HID), lambda i: (0, 0)),
            pl.BlockSpec((D_HID, 64), lambda i: (0, 0)),
            pl.BlockSpec((1, 64), lambda i: (0, 0)),
        ],
        out_specs=pl.BlockSpec((NUM_GRAPHS, 64), lambda i: (0, 0)),
        out_shape=jax.ShapeDtypeStruct((NUM_GRAPHS, 64), jnp.float32),
        scratch_shapes=[
            pltpu.VMEM((NUM_GRAPHS, D_HID), jnp.float32),
            pltpu.VMEM((NUM_GRAPHS, 1), jnp.float32),
        ],
    )(agg2d, sdst, batch3, wm1, bm1, wm2, bm2)


def kernel(x, edge_index, batch, W0, b0, W1, b1, W2, b2, Wm1, bm1, Wm2, bm2):
    sl = jnp.arange(N, dtype=jnp.int32)
    row = jnp.concatenate([edge_index[0], sl])
    col = jnp.concatenate([edge_index[1], sl])
    npad = E_PAD - E_TOT
    pad_dummy = jnp.full((npad,), DUMMY, jnp.int32)
    row_g = jnp.concatenate([row, jnp.zeros((npad,), jnp.int32)])
    row_h = jnp.concatenate([row, pad_dummy])
    col_p = jnp.concatenate([col, pad_dummy])

    hist_idx = jnp.stack([row_h.reshape(16, NCH, CHUNK),
                          col_p.reshape(16, NCH, CHUNK)])
    hist = _sc_hist(hist_idx)

    row4 = row_g.reshape(1, 16, NCH, CHUNK) * 4
    rowidx4 = row4 + jnp.arange(4, dtype=jnp.int32).reshape(4, 1, 1, 1)
    colidx = col_p.reshape(16, NCH, CHUNK)
    zeros_blk = jnp.zeros((CHUNK, FW), jnp.bfloat16)

    def agg(g2d):
        out = _sc_agg(g2d.reshape(4 * N, FW), rowidx4, colidx, zeros_blk)
        return out.reshape(NPAD, 4 * FW)

    g1, dis, sdst = _tc_in(x, W0, b0.reshape(1, D_HID), hist)
    g2 = _tc_mid(agg(g1), sdst, dis, W1, b1.reshape(1, D_HID))
    g3 = _tc_mid(agg(g2), sdst, dis, W2, b2.reshape(1, D_HID))
    agg3 = agg(g3)
    batch3 = batch.reshape(NBLK, 1, BN)
    return _tc_fin(agg3, sdst, batch3, Wm1, bm1.reshape(1, D_HID),
                   Wm2, bm2.reshape(1, 64))


# speedup vs baseline: 18.6145x; 1.1023x over previous
import functools

import jax
import jax.numpy as jnp
from jax import lax
from jax.experimental import pallas as pl
from jax.experimental.pallas import tpu as pltpu
from jax.experimental.pallas import tpu_sc as plsc

N = 10000
NUM_GRAPHS = 64
E = 320000
E_TOT = E + N
CHUNK = 128
NCH = 162
E_PAD = 16 * NCH * CHUNK
DUMMY = 10100
NPAD = 10240
ROWS_PER_TILE = NPAD // 16
BN = 400
NBLK = N // BN
FW = 128
D_HID = 256

_mesh = plsc.VectorSubcoreMesh(core_axis_name="c", subcore_axis_name="s",
                               num_cores=2, num_subcores=16)


@functools.partial(
    pl.kernel,
    out_type=jax.ShapeDtypeStruct((2, NPAD, 16), jnp.float32),
    mesh=_mesh,
    scratch_types=[
        pltpu.VMEM((NCH, CHUNK), jnp.int32),
        pltpu.VMEM((CHUNK, 16), jnp.float32),
        pltpu.VMEM((ROWS_PER_TILE, 16), jnp.float32),
        pltpu.VMEM_SHARED((NPAD, 16), jnp.float32),
    ],
    compiler_params=pltpu.CompilerParams(use_tc_tiling_on_sc=False),
)
def _sc_hist(idx_hbm, out_hbm, idx_v, ones_v, buf_v, acc_sh):
    c = lax.axis_index("c")
    s = lax.axis_index("s")
    one = jnp.full((16,), 1.0, jnp.float32)
    zero = jnp.zeros((16,), jnp.float32)

    def fill_ones(i, _):
        ones_v[i] = one
        return 0

    lax.fori_loop(0, CHUNK, fill_ones, 0)

    def fill_zero(i, _):
        buf_v[i] = zero
        return 0

    lax.fori_loop(0, ROWS_PER_TILE, fill_zero, 0)
    pltpu.sync_copy(buf_v, acc_sh.at[pl.ds(s * ROWS_PER_TILE, ROWS_PER_TILE)])
    plsc.subcore_barrier()

    pltpu.sync_copy(idx_hbm.at[c].at[s], idx_v)

    def body(j, _):
        pltpu.sync_copy(ones_v, acc_sh.at[idx_v.at[j]], add=True)
        return 0

    lax.fori_loop(0, NCH, body, 0)
    plsc.subcore_barrier()

    pltpu.sync_copy(acc_sh.at[pl.ds(s * ROWS_PER_TILE, ROWS_PER_TILE)], buf_v)
    pltpu.sync_copy(buf_v, out_hbm.at[c].at[pl.ds(s * ROWS_PER_TILE, ROWS_PER_TILE)])


@functools.partial(
    pl.kernel,
    out_type=jax.ShapeDtypeStruct((NPAD, 2, FW), jnp.bfloat16),
    mesh=_mesh,
    scratch_types=[
        pltpu.VMEM((1, NCH, CHUNK), jnp.int32),
        pltpu.VMEM((NCH, CHUNK), jnp.int32),
        pltpu.VMEM((3, CHUNK, FW), jnp.bfloat16),
        pltpu.VMEM_SHARED((NPAD, FW), jnp.bfloat16),
        pltpu.SemaphoreType.DMA,
        pltpu.SemaphoreType.DMA,
    ],
    compiler_params=pltpu.CompilerParams(use_tc_tiling_on_sc=False),
)
def _sc_agg(g4_hbm, rowidx4_hbm, colidx_hbm, zeros_hbm, out_hbm,
            ridx_v, cidx_v, bufs_v, acc_sh, semg1, sems1):
    semg = [semg1] * 3
    sems = [sems1] * 3
    c = lax.axis_index("c")
    s = lax.axis_index("s")
    base = s * ROWS_PER_TILE
    NB = 3
    NROUND = NCH // NB
    REM = NCH - NROUND * NB

    pltpu.sync_copy(colidx_hbm.at[s], cidx_v)

    for p in range(1):
        q = c
        pltpu.sync_copy(rowidx4_hbm.at[q].at[s], ridx_v.at[p])

        pltpu.sync_copy(zeros_hbm, bufs_v.at[0])
        for k in range(ROWS_PER_TILE // CHUNK):
            pltpu.async_copy(bufs_v.at[0],
                             acc_sh.at[pl.ds(base + k * CHUNK, CHUNK)], sems[0])
        for k in range(ROWS_PER_TILE // CHUNK):
            pltpu.make_async_copy(
                bufs_v.at[0],
                acc_sh.at[pl.ds(base + k * CHUNK, CHUNK)], sems[0]).wait()
        plsc.subcore_barrier()

        ridx_p = ridx_v.at[p]

        def gather(j, b):
            pltpu.async_copy(g4_hbm.at[ridx_p.at[j]], bufs_v.at[b], semg[b])

        def gather_wait(j, b):
            pltpu.make_async_copy(g4_hbm.at[ridx_p.at[j]], bufs_v.at[b],
                                  semg[b]).wait()

        def scatter(j, b):
            pltpu.async_copy(bufs_v.at[b], acc_sh.at[cidx_v.at[j]], sems[b],
                             add=True)

        def scatter_wait(j, b):
            pltpu.make_async_copy(bufs_v.at[b], acc_sh.at[cidx_v.at[j]],
                                  sems[b]).wait()

        for b in range(NB):
            gather(b, b)

        def body(jj, _):
            j = jj * NB
            for b in range(NB):
                gather_wait(j + b, b)
                scatter(j + b, b)
            for b in range(NB):
                scatter_wait(j + b, b)
                nxt = jnp.minimum(j + NB + b, NCH - 1)
                gather(nxt, b)
            return 0

        lax.fori_loop(0, NROUND, body, 0)
        for b in range(NB):
            j = NROUND * NB + b
            if b < REM:
                gather_wait(j, b)
                scatter(j, b)
                scatter_wait(j, b)
            else:
                gather_wait(NCH - 1, b)
        plsc.subcore_barrier()

        NK = ROWS_PER_TILE // CHUNK
        for k in range(NK):
            b = k % NB
            if k >= NB:
                pltpu.make_async_copy(
                    bufs_v.at[b],
                    out_hbm.at[pl.ds(base + (k - NB) * CHUNK, CHUNK), q],
                    sems[b]).wait()
            pltpu.async_copy(acc_sh.at[pl.ds(base + k * CHUNK, CHUNK)],
                             bufs_v.at[b], semg[b])
            pltpu.make_async_copy(acc_sh.at[pl.ds(base + k * CHUNK, CHUNK)],
                                  bufs_v.at[b], semg[b]).wait()
            pltpu.async_copy(bufs_v.at[b],
                             out_hbm.at[pl.ds(base + k * CHUNK, CHUNK), q],
                             sems[b])
        for k in range(max(NK - NB, 0), NK):
            b = k % NB
            pltpu.make_async_copy(
                bufs_v.at[b],
                out_hbm.at[pl.ds(base + k * CHUNK, CHUNK), q], sems[b]).wait()
        plsc.subcore_barrier()


def _tc_in_body(x_ref, w_ref, b_ref, hist_ref, g_ref, dis_ref, sdst_ref):
    deg = hist_ref[0, :, 0:1]
    cnt = hist_ref[1, :, 0:1]
    dis = lax.rsqrt(jnp.maximum(deg, 1.0))
    lin = jnp.dot(x_ref[...], w_ref[...], preferred_element_type=jnp.float32)
    g_ref[...] = (dis * (lin + b_ref[...])).astype(jnp.bfloat16)
    dis_ref[...] = dis
    sdst_ref[...] = dis / jnp.maximum(cnt, 1.0)


def _tc_in(x, w, b, hist):
    return pl.pallas_call(
        _tc_in_body,
        grid=(NBLK,),
        in_specs=[
            pl.BlockSpec((BN, 128), lambda i: (i, 0)),
            pl.BlockSpec((128, D_HID), lambda i: (0, 0)),
            pl.BlockSpec((1, D_HID), lambda i: (0, 0)),
            pl.BlockSpec((2, BN, 16), lambda i: (0, i, 0)),
        ],
        out_specs=[
            pl.BlockSpec((BN, D_HID), lambda i: (i, 0)),
            pl.BlockSpec((BN, 1), lambda i: (i, 0)),
            pl.BlockSpec((BN, 1), lambda i: (i, 0)),
        ],
        out_shape=[
            jax.ShapeDtypeStruct((N, D_HID), jnp.bfloat16),
            jax.ShapeDtypeStruct((N, 1), jnp.float32),
            jax.ShapeDtypeStruct((N, 1), jnp.float32),
        ],
    )(x, w, b, hist)


def _tc_mid_body(agg_ref, sdst_ref, dis_ref, w_ref, b_ref, g_ref):
    a = agg_ref[...].astype(jnp.float32)
    h = jnp.maximum(sdst_ref[...] * a, 0.0)
    lin = jnp.dot(h, w_ref[...], preferred_element_type=jnp.float32)
    g_ref[...] = (dis_ref[...] * (lin + b_ref[...])).astype(jnp.bfloat16)


def _tc_mid(agg2d, sdst, dis, w, b):
    return pl.pallas_call(
        _tc_mid_body,
        grid=(NBLK,),
        in_specs=[
            pl.BlockSpec((BN, D_HID), lambda i: (i, 0)),
            pl.BlockSpec((BN, 1), lambda i: (i, 0)),
            pl.BlockSpec((BN, 1), lambda i: (i, 0)),
            pl.BlockSpec((D_HID, D_HID), lambda i: (0, 0)),
            pl.BlockSpec((1, D_HID), lambda i: (0, 0)),
        ],
        out_specs=pl.BlockSpec((BN, D_HID), lambda i: (i, 0)),
        out_shape=jax.ShapeDtypeStruct((N, D_HID), jnp.bfloat16),
    )(agg2d, sdst, dis, w, b)


def _tc_fin_body(agg_ref, sdst_ref, batch_ref, wm1_ref, bm1_ref,
                 wm2_ref, bm2_ref, out_ref, pool_acc, cnt_acc):
    i = pl.program_id(0)
    h = jnp.maximum(sdst_ref[...] * agg_ref[...].astype(jnp.float32), 0.0)
    bt = batch_ref[0]
    oh = (bt == lax.broadcasted_iota(jnp.int32, (NUM_GRAPHS, BN), 0))
    oh = oh.astype(jnp.float32)

    @pl.when(i == 0)
    def _():
        pool_acc[...] = jnp.zeros_like(pool_acc)
        cnt_acc[...] = jnp.zeros_like(cnt_acc)

    pool_acc[...] += jnp.dot(oh, h, preferred_element_type=jnp.float32)
    cnt_acc[...] += jnp.sum(oh, axis=1, keepdims=True)

    @pl.when(i == NBLK - 1)
    def _():
        pooled = pool_acc[...] / jnp.maximum(cnt_acc[...], 1.0)
        z = jnp.dot(pooled, wm1_ref[...], preferred_element_type=jnp.float32)
        z = jnp.maximum(z + bm1_ref[...], 0.0)
        out_ref[...] = (
            jnp.dot(z, wm2_ref[...], preferred_element_type=jnp.float32)
            + bm2_ref[...]
        )


def _tc_fin(agg2d, sdst, batch3, wm1, bm1, wm2, bm2):
    return pl.pallas_call(
        _tc_fin_body,
        grid=(NBLK,),
        in_specs=[
            pl.BlockSpec((BN, D_HID), lambda i: (i, 0)),
            pl.BlockSpec((BN, 1), lambda i: (i, 0)),
            pl.BlockSpec((1, 1, BN), lambda i: (i, 0, 0)),
            pl.BlockSpec((D_HID, D_HID), lambda i: (0, 0)),
            pl.BlockSpec((1, D_HID), lambda i: (0, 0)),
            pl.BlockSpec((D_HID, 64), lambda i: (0, 0)),
            pl.BlockSpec((1, 64), lambda i: (0, 0)),
        ],
        out_specs=pl.BlockSpec((NUM_GRAPHS, 64), lambda i: (0, 0)),
        out_shape=jax.ShapeDtypeStruct((NUM_GRAPHS, 64), jnp.float32),
        scratch_shapes=[
            pltpu.VMEM((NUM_GRAPHS, D_HID), jnp.float32),
            pltpu.VMEM((NUM_GRAPHS, 1), jnp.float32),
        ],
    )(agg2d, sdst, batch3, wm1, bm1, wm2, bm2)


def kernel(x, edge_index, batch, W0, b0, W1, b1, W2, b2, Wm1, bm1, Wm2, bm2):
    sl = jnp.arange(N, dtype=jnp.int32)
    row = jnp.concatenate([edge_index[0], sl])
    col = jnp.concatenate([edge_index[1], sl])
    npad = E_PAD - E_TOT
    pad_dummy = jnp.full((npad,), DUMMY, jnp.int32)
    row_g = jnp.concatenate([row, jnp.zeros((npad,), jnp.int32)])
    row_h = jnp.concatenate([row, pad_dummy])
    col_p = jnp.concatenate([col, pad_dummy])

    hist_idx = jnp.stack([row_h.reshape(16, NCH, CHUNK),
                          col_p.reshape(16, NCH, CHUNK)])
    hist = _sc_hist(hist_idx)

    row4 = row_g.reshape(1, 16, NCH, CHUNK) * 2
    rowidx4 = row4 + jnp.arange(2, dtype=jnp.int32).reshape(2, 1, 1, 1)
    colidx = col_p.reshape(16, NCH, CHUNK)
    zeros_blk = jnp.zeros((CHUNK, FW), jnp.bfloat16)

    def agg(g2d):
        out = _sc_agg(g2d.reshape(2 * N, FW), rowidx4, colidx, zeros_blk)
        return out.reshape(NPAD, 2 * FW)

    g1, dis, sdst = _tc_in(x, W0, b0.reshape(1, D_HID), hist)
    g2 = _tc_mid(agg(g1), sdst, dis, W1, b1.reshape(1, D_HID))
    g3 = _tc_mid(agg(g2), sdst, dis, W2, b2.reshape(1, D_HID))
    agg3 = agg(g3)
    batch3 = batch.reshape(NBLK, 1, BN)
    return _tc_fin(agg3, sdst, batch3, Wm1, bm1.reshape(1, D_HID),
                   Wm2, bm2.reshape(1, 64))


# speedup vs baseline: 20.0593x; 1.0776x over previous
import functools

import jax
import jax.numpy as jnp
from jax import lax
from jax.experimental import pallas as pl
from jax.experimental.pallas import tpu as pltpu
from jax.experimental.pallas import tpu_sc as plsc

N = 10000
NUM_GRAPHS = 64
E = 320000
CHUNK = 128
NCH = 157
E_PAD = 16 * NCH * CHUNK
DUMMY = 10100
NPAD = 10240
ROWS_PER_TILE = NPAD // 16
BN = 400
NBLK = N // BN
FW = 128
D_HID = 256

_mesh = plsc.VectorSubcoreMesh(core_axis_name="c", subcore_axis_name="s",
                               num_cores=2, num_subcores=16)


@functools.partial(
    pl.kernel,
    out_type=jax.ShapeDtypeStruct((2, NPAD, 16), jnp.float32),
    mesh=_mesh,
    scratch_types=[
        pltpu.VMEM((NCH, CHUNK), jnp.int32),
        pltpu.VMEM((CHUNK, 16), jnp.float32),
        pltpu.VMEM((ROWS_PER_TILE, 16), jnp.float32),
        pltpu.VMEM_SHARED((NPAD, 16), jnp.float32),
    ],
    compiler_params=pltpu.CompilerParams(use_tc_tiling_on_sc=False),
)
def _sc_hist(idx_hbm, out_hbm, idx_v, ones_v, buf_v, acc_sh):
    c = lax.axis_index("c")
    s = lax.axis_index("s")
    one = jnp.full((16,), 1.0, jnp.float32)
    zero = jnp.zeros((16,), jnp.float32)

    def fill_ones(i, _):
        ones_v[i] = one
        return 0

    lax.fori_loop(0, CHUNK, fill_ones, 0)

    def fill_zero(i, _):
        buf_v[i] = zero
        return 0

    lax.fori_loop(0, ROWS_PER_TILE, fill_zero, 0)
    pltpu.sync_copy(buf_v, acc_sh.at[pl.ds(s * ROWS_PER_TILE, ROWS_PER_TILE)])
    plsc.subcore_barrier()

    pltpu.sync_copy(idx_hbm.at[c].at[s], idx_v)

    def body(j, _):
        pltpu.sync_copy(ones_v, acc_sh.at[idx_v.at[j]], add=True)
        return 0

    lax.fori_loop(0, NCH, body, 0)
    plsc.subcore_barrier()

    pltpu.sync_copy(acc_sh.at[pl.ds(s * ROWS_PER_TILE, ROWS_PER_TILE)], buf_v)
    pltpu.sync_copy(buf_v, out_hbm.at[c].at[pl.ds(s * ROWS_PER_TILE, ROWS_PER_TILE)])


@functools.partial(
    pl.kernel,
    out_type=jax.ShapeDtypeStruct((NPAD, 2, FW), jnp.bfloat16),
    mesh=_mesh,
    scratch_types=[
        pltpu.VMEM((1, NCH, CHUNK), jnp.int32),
        pltpu.VMEM((NCH, CHUNK), jnp.int32),
        pltpu.VMEM((3, CHUNK, FW), jnp.bfloat16),
        pltpu.VMEM_SHARED((NPAD, FW), jnp.bfloat16),
        pltpu.SemaphoreType.DMA,
        pltpu.SemaphoreType.DMA,
    ],
    compiler_params=pltpu.CompilerParams(use_tc_tiling_on_sc=False),
)
def _sc_agg(g4_hbm, rowidx4_hbm, colidx_hbm, zeros_hbm, out_hbm,
            ridx_v, cidx_v, bufs_v, acc_sh, semg1, sems1):
    semg = [semg1] * 3
    sems = [sems1] * 3
    c = lax.axis_index("c")
    s = lax.axis_index("s")
    base = s * ROWS_PER_TILE
    NB = 3
    NROUND = NCH // NB
    REM = NCH - NROUND * NB

    pltpu.sync_copy(colidx_hbm.at[s], cidx_v)

    for p in range(1):
        q = c
        pltpu.sync_copy(rowidx4_hbm.at[q].at[s], ridx_v.at[p])

        pltpu.sync_copy(zeros_hbm, bufs_v.at[0])
        for k in range(ROWS_PER_TILE // CHUNK):
            pltpu.async_copy(bufs_v.at[0],
                             acc_sh.at[pl.ds(base + k * CHUNK, CHUNK)], sems[0])
        for k in range(ROWS_PER_TILE // CHUNK):
            pltpu.make_async_copy(
                bufs_v.at[0],
                acc_sh.at[pl.ds(base + k * CHUNK, CHUNK)], sems[0]).wait()
        plsc.subcore_barrier()

        ridx_p = ridx_v.at[p]

        def gather(j, b):
            pltpu.async_copy(g4_hbm.at[ridx_p.at[j]], bufs_v.at[b], semg[b])

        def gather_wait(j, b):
            pltpu.make_async_copy(g4_hbm.at[ridx_p.at[j]], bufs_v.at[b],
                                  semg[b]).wait()

        def scatter(j, b):
            pltpu.async_copy(bufs_v.at[b], acc_sh.at[cidx_v.at[j]], sems[b],
                             add=True)

        def scatter_wait(j, b):
            pltpu.make_async_copy(bufs_v.at[b], acc_sh.at[cidx_v.at[j]],
                                  sems[b]).wait()

        for b in range(NB):
            gather(b, b)

        def body(jj, _):
            j = jj * NB
            for b in range(NB):
                gather_wait(j + b, b)
                scatter(j + b, b)
            for b in range(NB):
                scatter_wait(j + b, b)
                nxt = jnp.minimum(j + NB + b, NCH - 1)
                gather(nxt, b)
            return 0

        lax.fori_loop(0, NROUND, body, 0)
        for b in range(NB):
            j = NROUND * NB + b
            if b < REM:
                gather_wait(j, b)
                scatter(j, b)
                scatter_wait(j, b)
            else:
                gather_wait(NCH - 1, b)
        plsc.subcore_barrier()

        NK = ROWS_PER_TILE // CHUNK
        for k in range(NK):
            b = k % NB
            if k >= NB:
                pltpu.make_async_copy(
                    bufs_v.at[b],
                    out_hbm.at[pl.ds(base + (k - NB) * CHUNK, CHUNK), q],
                    sems[b]).wait()
            pltpu.async_copy(acc_sh.at[pl.ds(base + k * CHUNK, CHUNK)],
                             bufs_v.at[b], semg[b])
            pltpu.make_async_copy(acc_sh.at[pl.ds(base + k * CHUNK, CHUNK)],
                                  bufs_v.at[b], semg[b]).wait()
            pltpu.async_copy(bufs_v.at[b],
                             out_hbm.at[pl.ds(base + k * CHUNK, CHUNK), q],
                             sems[b])
        for k in range(max(NK - NB, 0), NK):
            b = k % NB
            pltpu.make_async_copy(
                bufs_v.at[b],
                out_hbm.at[pl.ds(base + k * CHUNK, CHUNK), q], sems[b]).wait()
        plsc.subcore_barrier()


def _tc_in_body(x_ref, w_ref, b_ref, hist_ref, g_ref, dis_ref, sdst_ref):
    deg = hist_ref[0, :, 0:1] + 1.0
    cnt = hist_ref[1, :, 0:1] + 1.0
    dis = lax.rsqrt(deg)
    lin = jnp.dot(x_ref[...], w_ref[...], preferred_element_type=jnp.float32)
    g_ref[...] = (dis * (lin + b_ref[...])).astype(jnp.bfloat16)
    dis_ref[...] = dis
    sdst_ref[...] = dis / cnt


def _tc_in(x, w, b, hist):
    return pl.pallas_call(
        _tc_in_body,
        grid=(NBLK,),
        in_specs=[
            pl.BlockSpec((BN, 128), lambda i: (i, 0)),
            pl.BlockSpec((128, D_HID), lambda i: (0, 0)),
            pl.BlockSpec((1, D_HID), lambda i: (0, 0)),
            pl.BlockSpec((2, BN, 16), lambda i: (0, i, 0)),
        ],
        out_specs=[
            pl.BlockSpec((BN, D_HID), lambda i: (i, 0)),
            pl.BlockSpec((BN, 1), lambda i: (i, 0)),
            pl.BlockSpec((BN, 1), lambda i: (i, 0)),
        ],
        out_shape=[
            jax.ShapeDtypeStruct((N, D_HID), jnp.bfloat16),
            jax.ShapeDtypeStruct((N, 1), jnp.float32),
            jax.ShapeDtypeStruct((N, 1), jnp.float32),
        ],
    )(x, w, b, hist)


def _tc_mid_body(agg_ref, gin_ref, sdst_ref, dis_ref, w_ref, b_ref, g_ref):
    a = agg_ref[...].astype(jnp.float32) + gin_ref[...].astype(jnp.float32)
    h = jnp.maximum(sdst_ref[...] * a, 0.0)
    lin = jnp.dot(h, w_ref[...], preferred_element_type=jnp.float32)
    g_ref[...] = (dis_ref[...] * (lin + b_ref[...])).astype(jnp.bfloat16)


def _tc_mid(agg2d, gin, sdst, dis, w, b):
    return pl.pallas_call(
        _tc_mid_body,
        grid=(NBLK,),
        in_specs=[
            pl.BlockSpec((BN, D_HID), lambda i: (i, 0)),
            pl.BlockSpec((BN, D_HID), lambda i: (i, 0)),
            pl.BlockSpec((BN, 1), lambda i: (i, 0)),
            pl.BlockSpec((BN, 1), lambda i: (i, 0)),
            pl.BlockSpec((D_HID, D_HID), lambda i: (0, 0)),
            pl.BlockSpec((1, D_HID), lambda i: (0, 0)),
        ],
        out_specs=pl.BlockSpec((BN, D_HID), lambda i: (i, 0)),
        out_shape=jax.ShapeDtypeStruct((N, D_HID), jnp.bfloat16),
    )(agg2d, gin, sdst, dis, w, b)


def _tc_fin_body(agg_ref, gin_ref, sdst_ref, batch_ref, wm1_ref, bm1_ref,
                 wm2_ref, bm2_ref, out_ref, pool_acc, cnt_acc):
    i = pl.program_id(0)
    a = agg_ref[...].astype(jnp.float32) + gin_ref[...].astype(jnp.float32)
    h = jnp.maximum(sdst_ref[...] * a, 0.0)
    bt = batch_ref[0]
    oh = (bt == lax.broadcasted_iota(jnp.int32, (NUM_GRAPHS, BN), 0))
    oh = oh.astype(jnp.float32)

    @pl.when(i == 0)
    def _():
        pool_acc[...] = jnp.zeros_like(pool_acc)
        cnt_acc[...] = jnp.zeros_like(cnt_acc)

    pool_acc[...] += jnp.dot(oh, h, preferred_element_type=jnp.float32)
    cnt_acc[...] += jnp.sum(oh, axis=1, keepdims=True)

    @pl.when(i == NBLK - 1)
    def _():
        pooled = pool_acc[...] / jnp.maximum(cnt_acc[...], 1.0)
        z = jnp.dot(pooled, wm1_ref[...], preferred_element_type=jnp.float32)
        z = jnp.maximum(z + bm1_ref[...], 0.0)
        out_ref[...] = (
            jnp.dot(z, wm2_ref[...], preferred_element_type=jnp.float32)
            + bm2_ref[...]
        )


def _tc_fin(agg2d, gin, sdst, batch3, wm1, bm1, wm2, bm2):
    return pl.pallas_call(
        _tc_fin_body,
        grid=(NBLK,),
        in_specs=[
            pl.BlockSpec((BN, D_HID), lambda i: (i, 0)),
            pl.BlockSpec((BN, D_HID), lambda i: (i, 0)),
            pl.BlockSpec((BN, 1), lambda i: (i, 0)),
            pl.BlockSpec((1, 1, BN), lambda i: (i, 0, 0)),
            pl.BlockSpec((D_HID, D_HID), lambda i: (0, 0)),
            pl.BlockSpec((1, D_HID), lambda i: (0, 0)),
            pl.BlockSpec((D_HID, 64), lambda i: (0, 0)),
            pl.BlockSpec((1, 64), lambda i: (0, 0)),
        ],
        out_specs=pl.BlockSpec((NUM_GRAPHS, 64), lambda i: (0, 0)),
        out_shape=jax.ShapeDtypeStruct((NUM_GRAPHS, 64), jnp.float32),
        scratch_shapes=[
            pltpu.VMEM((NUM_GRAPHS, D_HID), jnp.float32),
            pltpu.VMEM((NUM_GRAPHS, 1), jnp.float32),
        ],
    )(agg2d, gin, sdst, batch3, wm1, bm1, wm2, bm2)


def kernel(x, edge_index, batch, W0, b0, W1, b1, W2, b2, Wm1, bm1, Wm2, bm2):
    npad = E_PAD - E
    pad_dummy = jnp.full((npad,), DUMMY, jnp.int32)
    row_g = jnp.concatenate([edge_index[0], jnp.zeros((npad,), jnp.int32)])
    row_h = jnp.concatenate([edge_index[0], pad_dummy])
    col_p = jnp.concatenate([edge_index[1], pad_dummy])

    hist_idx = jnp.stack([row_h.reshape(16, NCH, CHUNK),
                          col_p.reshape(16, NCH, CHUNK)])
    hist = _sc_hist(hist_idx)

    row4 = row_g.reshape(1, 16, NCH, CHUNK) * 2
    rowidx4 = row4 + jnp.arange(2, dtype=jnp.int32).reshape(2, 1, 1, 1)
    colidx = col_p.reshape(16, NCH, CHUNK)
    zeros_blk = jnp.zeros((CHUNK, FW), jnp.bfloat16)

    def agg(g2d):
        out = _sc_agg(g2d.reshape(2 * N, FW), rowidx4, colidx, zeros_blk)
        return out.reshape(NPAD, 2 * FW)

    g1, dis, sdst = _tc_in(x, W0, b0.reshape(1, D_HID), hist)
    g2 = _tc_mid(agg(g1), g1, sdst, dis, W1, b1.reshape(1, D_HID))
    g3 = _tc_mid(agg(g2), g2, sdst, dis, W2, b2.reshape(1, D_HID))
    agg3 = agg(g3)
    batch3 = batch.reshape(NBLK, 1, BN)
    return _tc_fin(agg3, g3, sdst, batch3, Wm1, bm1.reshape(1, D_HID),
                   Wm2, bm2.reshape(1, 64))
